# baseline (speedup 1.0000x reference)
import functools

import jax
import jax.numpy as jnp
from jax import lax
from jax.experimental import pallas as pl
from jax.experimental.pallas import tpu as pltpu
from jax.experimental.pallas import tpu_sc as plsc

B = 4096
F = 26
V = 100000
D = 16

NC, NS = 2, 16
NW = NC * NS
BPW = B // NW
IPW = BPW * F
GCH = 128
NCHUNK = IPW // GCH

_mesh = plsc.VectorSubcoreMesh(core_axis_name="c", subcore_axis_name="s")


@functools.partial(
    pl.kernel,
    out_type=jax.ShapeDtypeStruct((B,), jnp.float32),
    mesh=_mesh,
    scratch_types=[
        pltpu.VMEM((IPW,), jnp.int32),
        pltpu.VMEM((NCHUNK, GCH), jnp.int32),
        pltpu.VMEM((IPW, D), jnp.float32),
        pltpu.VMEM((IPW * 8,), jnp.float32),
        pltpu.VMEM((IPW + D,), jnp.int32),
        pltpu.VMEM((BPW,), jnp.float32),
        pltpu.VMEM((16,), jnp.float32),
        pltpu.SemaphoreType.DMA,
        pltpu.SemaphoreType.DMA,
    ],
    compiler_params=pltpu.CompilerParams(
        needs_layout_passes=False, use_tc_tiling_on_sc=False),
)
def _fm_fwd(idx_hbm, wide_hbm, emb_hbm, bias_hbm, out_hbm,
            idx_v, idx2_v, rows_v, wv8_v, rm8_v, acc_v, bias_v,
            sem_e, sem_w):
    wid = lax.axis_index("s") * NC + lax.axis_index("c")
    base = wid * BPW

    pltpu.sync_copy(idx_hbm.at[pl.ds(base * F, IPW)], idx_v)
    pltpu.sync_copy(bias_hbm, bias_v)

    lanes = lax.iota(jnp.int32, 16)

    def chunk_body(c, carry):
        cbase = c * GCH
        for v in range(GCH // 16):
            off = cbase + v * 16
            pos = lanes + off
            fld = lax.rem(pos, F)
            t = idx_v[pl.ds(off, 16)] + fld * V
            idx2_v[c, pl.ds(v * 16, 16)] = t
            rm8_v[pl.ds(off, 16)] = pos * 8 + (t & 7)
            for l in range(16):
                r = t[l]
                j = off + l
                pltpu.async_copy(emb_hbm.at[r, :], rows_v.at[j, :], sem_e)
                pltpu.async_copy(
                    wide_hbm.at[pl.ds(pl.multiple_of(r & -8, 8), 8)],
                    wv8_v.at[pl.ds(pl.multiple_of(j * 8, 8), 8)], sem_w)

        @pl.when(c > 0)
        def _():
            pltpu.make_async_copy(
                emb_hbm.at[pl.ds(0, GCH), :],
                rows_v.at[pl.ds(0, GCH), :], sem_e).wait()
            pltpu.make_async_copy(
                wide_hbm.at[pl.ds(0, GCH * 8)],
                wv8_v.at[pl.ds(0, GCH * 8)], sem_w).wait()
        return carry

    lax.fori_loop(0, NCHUNK, chunk_body, 0)

    pltpu.make_async_copy(
        emb_hbm.at[pl.ds(0, GCH), :], rows_v.at[pl.ds(0, GCH), :],
        sem_e).wait()
    pltpu.make_async_copy(
        wide_hbm.at[pl.ds(0, GCH * 8)],
        wv8_v.at[pl.ds(0, GCH * 8)], sem_w).wait()

    mask10b = lanes < (F - 16)
    mask10 = jnp.where(mask10b, 1.0, 0.0).astype(jnp.float32)
    bias_vec = bias_v[...]
    zero16 = jnp.zeros((16,), jnp.float32)

    def grp_body(g, carry):
        gb = g * 16
        acc = zero16
        for l in range(16):
            rb = (gb + l) * F
            e0 = rows_v[rb, :]
            s = e0
            q = e0 * e0
            for f in range(1, F):
                e = rows_v[rb + f, :]
                s = s + e
                q = q + e * e
            fmv = s * s - q
            w1 = plsc.load_gather(wv8_v, [rm8_v[pl.ds(rb, 16)]])
            w2 = plsc.load_gather(wv8_v, [rm8_v[pl.ds(rb + 16, 16)]],
                                  mask=mask10b)
            t = 0.5 * fmv + w1 + w2 * mask10
            acc = jnp.where(lanes == l, jnp.sum(t), acc)
        x = acc + bias_vec
        acc_v[pl.ds(gb, 16)] = 1.0 / (1.0 + jnp.exp(-x))
        return carry

    lax.fori_loop(0, BPW // 16, grp_body, 0)

    pltpu.sync_copy(acc_v, out_hbm.at[pl.ds(base, BPW)])


def kernel(indices, wide_table, emb_table, bias):
    flat_ids = indices.reshape(B * F)
    bias16 = jnp.broadcast_to(bias, (16,))
    out = _fm_fwd(flat_ids, wide_table, emb_table, bias16)
    return out.reshape(B, 1)


# speedup vs baseline: 1.0160x; 1.0160x over previous
import functools

import jax
import jax.numpy as jnp
from jax import lax
from jax.experimental import pallas as pl
from jax.experimental.pallas import tpu as pltpu
from jax.experimental.pallas import tpu_sc as plsc

B = 4096
F = 26
V = 100000
D = 16

NC, NS = 2, 16
NW = NC * NS
BPW = B // NW
IPW = BPW * F
GCH = 128
NCHUNK = IPW // GCH

_mesh = plsc.VectorSubcoreMesh(core_axis_name="c", subcore_axis_name="s")


@functools.partial(
    pl.kernel,
    out_type=jax.ShapeDtypeStruct((B,), jnp.float32),
    mesh=_mesh,
    scratch_types=[
        pltpu.VMEM((IPW,), jnp.int32),
        pltpu.VMEM((NCHUNK, GCH), jnp.int32),
        pltpu.VMEM((IPW, D), jnp.float32),
        pltpu.VMEM((IPW + D,), jnp.float32),
        pltpu.VMEM((BPW,), jnp.float32),
        pltpu.VMEM((16,), jnp.float32),
        pltpu.SemaphoreType.DMA,
        pltpu.SemaphoreType.DMA,
    ],
    compiler_params=pltpu.CompilerParams(
        needs_layout_passes=False, use_tc_tiling_on_sc=False),
)
def _fm_fwd(idx_hbm, wide_hbm, emb_hbm, bias_hbm, out_hbm,
            idx_v, idx2_v, rows_v, wv_v, acc_v, bias_v, sem_e, sem_w):
    wid = lax.axis_index("s") * NC + lax.axis_index("c")
    base = wid * BPW

    pltpu.sync_copy(idx_hbm.at[pl.ds(base * F, IPW)], idx_v)
    pltpu.sync_copy(bias_hbm, bias_v)

    lanes = lax.iota(jnp.int32, 16)

    def chunk_body(c, carry):
        cbase = c * GCH
        for v in range(GCH // 16):
            off = cbase + v * 16
            pos = lanes + off
            fld = lax.rem(pos, F)
            idx2_v[c, pl.ds(v * 16, 16)] = idx_v[pl.ds(off, 16)] + fld * V
        csl = pl.ds(cbase, GCH)
        pltpu.async_copy(emb_hbm.at[idx2_v.at[c]], rows_v.at[csl, :], sem_e)
        pltpu.async_copy(wide_hbm.at[idx2_v.at[c]], wv_v.at[csl], sem_w)
        return carry

    lax.fori_loop(0, NCHUNK, chunk_body, 0)

    pltpu.make_async_copy(
        emb_hbm.at[pl.ds(0, IPW), :], rows_v, sem_e).wait()
    pltpu.make_async_copy(
        wide_hbm.at[pl.ds(0, IPW)], wv_v.at[pl.ds(0, IPW)], sem_w).wait()

    mask10 = jnp.where(lanes < (F - 16), 1.0, 0.0).astype(jnp.float32)
    bias_vec = bias_v[...]
    zero16 = jnp.zeros((16,), jnp.float32)

    def grp_body(g, carry):
        gb = g * 16
        acc = zero16
        for l in range(16):
            rb = (gb + l) * F
            e0 = rows_v[rb, :]
            s = e0
            q = e0 * e0
            for f in range(1, F):
                e = rows_v[rb + f, :]
                s = s + e
                q = q + e * e
            fmv = s * s - q
            w1 = wv_v[pl.ds(rb, 16)]
            w2 = wv_v[pl.ds(rb + 16, 16)]
            t = 0.5 * fmv + w1 + w2 * mask10
            acc = jnp.where(lanes == l, jnp.sum(t), acc)
        x = acc + bias_vec
        acc_v[pl.ds(gb, 16)] = 1.0 / (1.0 + jnp.exp(-x))
        return carry

    lax.fori_loop(0, BPW // 16, grp_body, 0)

    pltpu.sync_copy(acc_v, out_hbm.at[pl.ds(base, BPW)])


def kernel(indices, wide_table, emb_table, bias):
    flat_ids = indices.reshape(B * F)
    bias16 = jnp.broadcast_to(bias, (16,))
    out = _fm_fwd(flat_ids, wide_table, emb_table, bias16)
    return out.reshape(B, 1)
